# Initial kernel scaffold; baseline (speedup 1.0000x reference)
import jax
import jax.numpy as jnp
from jax.experimental import pallas as pl


def kernel(x, weight):
    raise NotImplementedError("write your pallas kernel here")


# speedup vs baseline: 2.0722x; 2.0722x over previous
import functools

import jax
import jax.numpy as jnp
from jax.experimental import pallas as pl


_N = 4608
_Q = 512
_E = 32
_R = 256


def _vq_body(x_ref, wt_ref, w_ref, qd_ref, qi_ref):
    xb = x_ref[...]
    wt = wt_ref[...]

    block_sums = []
    for g in range(4):
        sq = []
        for e in range(8):
            ee = 8 * g + e
            d = wt[ee, :][None, :] - xb[:, ee][:, None]
            sq.append(d * d)
        t0 = [sq[i] + sq[i + 4] for i in range(4)]
        t1 = [t0[0] + t0[2], t0[1] + t0[3]]
        block_sums.append(t1[0] + t1[1])
    dist = ((block_sums[0] + block_sums[1]) + block_sums[2]) + block_sums[3]

    m = jnp.min(dist, axis=1, keepdims=True)
    qiota = jax.lax.broadcasted_iota(jnp.int32, (_R, _Q), 1)
    idx = jnp.min(jnp.where(dist == m, qiota, _Q), axis=1)

    onehot = (qiota == idx[:, None]).astype(jnp.float32)
    qd_ref[...] = jax.lax.dot(onehot, w_ref[...],
                              precision=jax.lax.Precision.HIGHEST)
    qi_ref[0, 0, :] = idx


@jax.jit
def _vq(xf, wt, w):
    nb = _N // _R
    qd, qi = pl.pallas_call(
        _vq_body,
        grid=(nb,),
        in_specs=[
            pl.BlockSpec((_R, _E), lambda i: (i, 0)),
            pl.BlockSpec((_E, _Q), lambda i: (0, 0)),
            pl.BlockSpec((_Q, _E), lambda i: (0, 0)),
        ],
        out_specs=[
            pl.BlockSpec((_R, _E), lambda i: (i, 0)),
            pl.BlockSpec((1, 1, _R), lambda i: (i, 0, 0)),
        ],
        out_shape=[
            jax.ShapeDtypeStruct((_N, _E), jnp.float32),
            jax.ShapeDtypeStruct((nb, 1, _R), jnp.int32),
        ],
    )(xf, wt, w)
    return qd, qi


def kernel(x, weight):
    xf = x.reshape(_N, _E)
    qd, qi = _vq(xf, weight.T, weight)
    return qd.reshape(x.shape), qi.reshape(x.shape[:-1])
